# baseline (speedup 1.0000x reference)
import jax
import jax.numpy as jnp
import numpy as np
from jax.experimental import pallas as pl
from jax.experimental.pallas import tpu as pltpu

B, U, GH, GW, E, DX, H = 4, 8192, 32, 32, 128, 2, 8
S = GH * GW
DH = E // H
BU = 1024
NU = U // BU
INV_SQRT_DH = 1.0 / np.sqrt(DH)


def _head_mask():
    r = jax.lax.broadcasted_iota(jnp.int32, (E, E), 0) // DH
    c = jax.lax.broadcasted_iota(jnp.int32, (E, E), 1) // DH
    return (r == c).astype(jnp.float32)


def _encoder_kernel(xc_ref, z_ref, on_ref, lat_ref, wq_ref, wk_ref, wv_ref,
                    wo_ref, out_ref, qm_ref, eon_ref, von_ref, num_ref,
                    den_ref):
    u = pl.program_id(1)
    mhead = _head_mask()

    @pl.when(u == 0)
    def _init():
        qm = jnp.dot(lat_ref[...], wq_ref[...],
                     preferred_element_type=jnp.float32)
        qm_ref[...] = qm
        on = on_ref[0]
        kon = jnp.dot(on, wk_ref[...], preferred_element_type=jnp.float32)
        von_ref[...] = jnp.dot(on, wv_ref[...],
                               preferred_element_type=jnp.float32)
        son = jnp.dot(qm * kon, mhead,
                      preferred_element_type=jnp.float32) * INV_SQRT_DH
        eon_ref[...] = jnp.exp(son)
        num_ref[...] = jnp.zeros_like(num_ref)
        den_ref[...] = jnp.zeros_like(den_ref)

    xc = xc_ref[0]
    z = z_ref[0]
    gi = jnp.clip(jnp.floor(xc[:, 0:1] * (GH - 1) + 0.5), 0, GH - 1)
    gj = jnp.clip(jnp.floor(xc[:, 1:2] * (GW - 1) + 0.5), 0, GW - 1)
    idx = (gi * GW + gj).astype(jnp.int32)
    onehot = (idx == jax.lax.broadcasted_iota(jnp.int32, (BU, S), 1)
              ).astype(jnp.float32)

    k = jnp.dot(z, wk_ref[...], preferred_element_type=jnp.float32)
    v = jnp.dot(z, wv_ref[...], preferred_element_type=jnp.float32)
    qg = jnp.dot(onehot, qm_ref[...], preferred_element_type=jnp.float32)
    scores = jnp.dot(qg * k, mhead,
                     preferred_element_type=jnp.float32) * INV_SQRT_DH
    w = jnp.exp(scores)

    contract0 = (((0,), (0,)), ((), ()))
    onehot_b = onehot.astype(jnp.bfloat16)
    num_ref[...] += jax.lax.dot_general(
        onehot_b, (v * w).astype(jnp.bfloat16), contract0,
        preferred_element_type=jnp.float32)
    den_ref[...] += jax.lax.dot_general(
        onehot_b, w.astype(jnp.bfloat16), contract0,
        preferred_element_type=jnp.float32)

    @pl.when(u == NU - 1)
    def _finalize():
        eon = eon_ref[...]
        outm = (num_ref[...] + eon * von_ref[...]) / (den_ref[...] + eon)
        out_ref[0] = jnp.dot(outm, wo_ref[...],
                             preferred_element_type=jnp.float32)


def kernel(xc_off_grid, xc_on_grid, zc_off_grid, zc_on_grid, ignore_on_grid,
           latents, fake_embedding, Wq, Wk, Wv, Wo):
    Bv = xc_on_grid.shape[0]
    grid_shape = xc_on_grid.shape[1:-1]
    zc_on = zc_on_grid.reshape(Bv, S, E)
    on_tok = jnp.where(jnp.asarray(ignore_on_grid),
                       jnp.broadcast_to(fake_embedding, (Bv, S, E)), zc_on)

    out = pl.pallas_call(
        _encoder_kernel,
        grid=(Bv, NU),
        in_specs=[
            pl.BlockSpec((1, BU, DX), lambda b, u: (b, u, 0)),
            pl.BlockSpec((1, BU, E), lambda b, u: (b, u, 0)),
            pl.BlockSpec((1, S, E), lambda b, u: (b, 0, 0)),
            pl.BlockSpec((S, E), lambda b, u: (0, 0)),
            pl.BlockSpec((E, E), lambda b, u: (0, 0)),
            pl.BlockSpec((E, E), lambda b, u: (0, 0)),
            pl.BlockSpec((E, E), lambda b, u: (0, 0)),
            pl.BlockSpec((E, E), lambda b, u: (0, 0)),
        ],
        out_specs=pl.BlockSpec((1, S, E), lambda b, u: (b, 0, 0)),
        out_shape=jax.ShapeDtypeStruct((Bv, S, E), jnp.float32),
        scratch_shapes=[
            pltpu.VMEM((S, E), jnp.float32),
            pltpu.VMEM((S, E), jnp.float32),
            pltpu.VMEM((S, E), jnp.float32),
            pltpu.VMEM((S, E), jnp.float32),
            pltpu.VMEM((S, E), jnp.float32),
        ],
        compiler_params=pltpu.CompilerParams(
            dimension_semantics=("parallel", "arbitrary")),
    )(xc_off_grid, zc_off_grid, on_tok, latents, Wq, Wk, Wv, Wo)

    return out.reshape((Bv,) + tuple(grid_shape) + (E,))


# speedup vs baseline: 1.0039x; 1.0039x over previous
import jax
import jax.numpy as jnp
import numpy as np
from jax.experimental import pallas as pl
from jax.experimental.pallas import tpu as pltpu

B, U, GH, GW, E, DX, H = 4, 8192, 32, 32, 128, 2, 8
S = GH * GW
DH = E // H
BU = 1024
NU = U // BU
INV_SQRT_DH = 1.0 / np.sqrt(DH)


def _head_mask():
    r = jax.lax.broadcasted_iota(jnp.int32, (E, E), 0) // DH
    c = jax.lax.broadcasted_iota(jnp.int32, (E, E), 1) // DH
    return (r == c).astype(jnp.float32)


def _encoder_kernel(xc_ref, z_ref, on_ref, lat_ref, wq_ref, wk_ref, wv_ref,
                    wo_ref, out_ref, qm_ref, eon_ref, von_ref, num_ref,
                    den_ref):
    u = pl.program_id(1)
    mhead = _head_mask()

    @pl.when(u == 0)
    def _init():
        qm = jnp.dot(lat_ref[...], wq_ref[...],
                     preferred_element_type=jnp.float32)
        qm_ref[...] = qm
        on = on_ref[0]
        kon = jnp.dot(on, wk_ref[...], preferred_element_type=jnp.float32)
        von_ref[...] = jnp.dot(on, wv_ref[...],
                               preferred_element_type=jnp.float32)
        son = jnp.dot(qm * kon, mhead,
                      preferred_element_type=jnp.float32) * INV_SQRT_DH
        eon_ref[...] = jnp.exp(son)
        num_ref[...] = jnp.zeros_like(num_ref)
        den_ref[...] = jnp.zeros_like(den_ref)

    xc = xc_ref[0]
    z = z_ref[0]
    gi = jnp.clip(jnp.floor(xc[:, 0:1] * (GH - 1) + 0.5), 0, GH - 1)
    gj = jnp.clip(jnp.floor(xc[:, 1:2] * (GW - 1) + 0.5), 0, GW - 1)
    idx = (gi * GW + gj).astype(jnp.int32)
    onehot = (idx == jax.lax.broadcasted_iota(jnp.int32, (BU, S), 1)
              ).astype(jnp.float32)

    k = jnp.dot(z, wk_ref[...], preferred_element_type=jnp.float32)
    v = jnp.dot(z, wv_ref[...], preferred_element_type=jnp.float32)
    qg = jnp.dot(onehot, qm_ref[...], preferred_element_type=jnp.float32)
    scores = jnp.dot(qg * k, mhead,
                     preferred_element_type=jnp.float32) * INV_SQRT_DH
    w = jnp.exp(scores)

    contract0 = (((0,), (0,)), ((), ()))
    num_ref[...] += jax.lax.dot_general(
        onehot, v * w, contract0, preferred_element_type=jnp.float32)
    den_ref[...] += jax.lax.dot_general(
        onehot, w, contract0, preferred_element_type=jnp.float32)

    @pl.when(u == NU - 1)
    def _finalize():
        eon = eon_ref[...]
        outm = (num_ref[...] + eon * von_ref[...]) / (den_ref[...] + eon)
        out_ref[0] = jnp.dot(outm, wo_ref[...],
                             preferred_element_type=jnp.float32)


def kernel(xc_off_grid, xc_on_grid, zc_off_grid, zc_on_grid, ignore_on_grid,
           latents, fake_embedding, Wq, Wk, Wv, Wo):
    Bv = xc_on_grid.shape[0]
    grid_shape = xc_on_grid.shape[1:-1]
    zc_on = zc_on_grid.reshape(Bv, S, E)
    on_tok = jnp.where(jnp.asarray(ignore_on_grid),
                       jnp.broadcast_to(fake_embedding, (Bv, S, E)), zc_on)

    out = pl.pallas_call(
        _encoder_kernel,
        grid=(Bv, NU),
        in_specs=[
            pl.BlockSpec((1, BU, DX), lambda b, u: (b, u, 0)),
            pl.BlockSpec((1, BU, E), lambda b, u: (b, u, 0)),
            pl.BlockSpec((1, S, E), lambda b, u: (b, 0, 0)),
            pl.BlockSpec((S, E), lambda b, u: (0, 0)),
            pl.BlockSpec((E, E), lambda b, u: (0, 0)),
            pl.BlockSpec((E, E), lambda b, u: (0, 0)),
            pl.BlockSpec((E, E), lambda b, u: (0, 0)),
            pl.BlockSpec((E, E), lambda b, u: (0, 0)),
        ],
        out_specs=pl.BlockSpec((1, S, E), lambda b, u: (b, 0, 0)),
        out_shape=jax.ShapeDtypeStruct((Bv, S, E), jnp.float32),
        scratch_shapes=[
            pltpu.VMEM((S, E), jnp.float32),
            pltpu.VMEM((S, E), jnp.float32),
            pltpu.VMEM((S, E), jnp.float32),
            pltpu.VMEM((S, E), jnp.float32),
            pltpu.VMEM((S, E), jnp.float32),
        ],
        compiler_params=pltpu.CompilerParams(
            dimension_semantics=("parallel", "arbitrary")),
    )(xc_off_grid, zc_off_grid, on_tok, latents, Wq, Wk, Wv, Wo)

    return out.reshape((Bv,) + tuple(grid_shape) + (E,))


# speedup vs baseline: 1.1399x; 1.1355x over previous
import jax
import jax.numpy as jnp
import numpy as np
from jax.experimental import pallas as pl
from jax.experimental.pallas import tpu as pltpu

B, U, GH, GW, E, DX, H = 4, 8192, 32, 32, 128, 2, 8
S = GH * GW
DH = E // H
BU = 4096
NU = U // BU
INV_SQRT_DH = 1.0 / np.sqrt(DH)


def _head_mask():
    r = jax.lax.broadcasted_iota(jnp.int32, (E, E), 0) // DH
    c = jax.lax.broadcasted_iota(jnp.int32, (E, E), 1) // DH
    return (r == c).astype(jnp.float32)


def _encoder_kernel(xc_ref, xct_ref, z_ref, on_ref, lat_ref, wq_ref, wk_ref,
                    wv_ref, wo_ref, out_ref, qm_ref, eon_ref, von_ref,
                    num_ref, den_ref):
    u = pl.program_id(1)
    mhead = _head_mask()

    @pl.when(u == 0)
    def _init():
        qm = jnp.dot(lat_ref[...], wq_ref[...],
                     preferred_element_type=jnp.float32)
        qm_ref[...] = qm
        on = on_ref[0]
        kon = jnp.dot(on, wk_ref[...], preferred_element_type=jnp.float32)
        von_ref[...] = jnp.dot(on, wv_ref[...],
                               preferred_element_type=jnp.float32)
        son = jnp.dot(qm * kon, mhead,
                      preferred_element_type=jnp.float32) * INV_SQRT_DH
        eon_ref[...] = jnp.exp(son)
        num_ref[...] = jnp.zeros_like(num_ref)
        den_ref[...] = jnp.zeros_like(den_ref)

    xc = xc_ref[0]
    z = z_ref[0]
    gi = jnp.clip(jnp.floor(xc[:, 0:1] * (GH - 1) + 0.5), 0, GH - 1)
    gj = jnp.clip(jnp.floor(xc[:, 1:2] * (GW - 1) + 0.5), 0, GW - 1)
    idx = (gi * GW + gj).astype(jnp.int32)
    onehot = (idx == jax.lax.broadcasted_iota(jnp.int32, (BU, S), 1)
              ).astype(jnp.float32)

    xv = xct_ref[0]
    gi_r = jnp.clip(jnp.floor(xv[0:1, :] * (GH - 1) + 0.5), 0, GH - 1)
    gj_r = jnp.clip(jnp.floor(xv[1:2, :] * (GW - 1) + 0.5), 0, GW - 1)
    idx_r = (gi_r * GW + gj_r).astype(jnp.int32)
    onehot_t = (idx_r == jax.lax.broadcasted_iota(jnp.int32, (S, BU), 0)
                ).astype(jnp.float32)

    k = jnp.dot(z, wk_ref[...], preferred_element_type=jnp.float32)
    v = jnp.dot(z, wv_ref[...], preferred_element_type=jnp.float32)
    qg = jnp.dot(onehot, qm_ref[...], preferred_element_type=jnp.float32)
    scores = jnp.dot(qg * k, mhead,
                     preferred_element_type=jnp.float32) * INV_SQRT_DH
    w = jnp.exp(scores)

    num_ref[...] += jnp.dot(onehot_t, v * w, preferred_element_type=jnp.float32)
    den_ref[...] += jnp.dot(onehot_t, w, preferred_element_type=jnp.float32)

    @pl.when(u == NU - 1)
    def _finalize():
        eon = eon_ref[...]
        outm = (num_ref[...] + eon * von_ref[...]) / (den_ref[...] + eon)
        out_ref[0] = jnp.dot(outm, wo_ref[...],
                             preferred_element_type=jnp.float32)


def kernel(xc_off_grid, xc_on_grid, zc_off_grid, zc_on_grid, ignore_on_grid,
           latents, fake_embedding, Wq, Wk, Wv, Wo):
    Bv = xc_on_grid.shape[0]
    grid_shape = xc_on_grid.shape[1:-1]
    zc_on = zc_on_grid.reshape(Bv, S, E)
    on_tok = jnp.where(jnp.asarray(ignore_on_grid),
                       jnp.broadcast_to(fake_embedding, (Bv, S, E)), zc_on)

    out = pl.pallas_call(
        _encoder_kernel,
        grid=(Bv, NU),
        in_specs=[
            pl.BlockSpec((1, BU, DX), lambda b, u: (b, u, 0)),
            pl.BlockSpec((1, DX, BU), lambda b, u: (b, 0, u)),
            pl.BlockSpec((1, BU, E), lambda b, u: (b, u, 0)),
            pl.BlockSpec((1, S, E), lambda b, u: (b, 0, 0)),
            pl.BlockSpec((S, E), lambda b, u: (0, 0)),
            pl.BlockSpec((E, E), lambda b, u: (0, 0)),
            pl.BlockSpec((E, E), lambda b, u: (0, 0)),
            pl.BlockSpec((E, E), lambda b, u: (0, 0)),
            pl.BlockSpec((E, E), lambda b, u: (0, 0)),
        ],
        out_specs=pl.BlockSpec((1, S, E), lambda b, u: (b, 0, 0)),
        out_shape=jax.ShapeDtypeStruct((Bv, S, E), jnp.float32),
        scratch_shapes=[
            pltpu.VMEM((S, E), jnp.float32),
            pltpu.VMEM((S, E), jnp.float32),
            pltpu.VMEM((S, E), jnp.float32),
            pltpu.VMEM((S, E), jnp.float32),
            pltpu.VMEM((S, E), jnp.float32),
        ],
        compiler_params=pltpu.CompilerParams(
            dimension_semantics=("parallel", "arbitrary")),
    )(xc_off_grid, xc_off_grid.transpose(0, 2, 1), zc_off_grid, on_tok,
      latents, Wq, Wk, Wv, Wo)

    return out.reshape((Bv,) + tuple(grid_shape) + (E,))


# speedup vs baseline: 1.1552x; 1.0135x over previous
import jax
import jax.numpy as jnp
import numpy as np
from jax.experimental import pallas as pl
from jax.experimental.pallas import tpu as pltpu

B, U, GH, GW, E, DX, H = 4, 8192, 32, 32, 128, 2, 8
S = GH * GW
DH = E // H
BU = 2048
NU = U // BU
INV_SQRT_DH = 1.0 / np.sqrt(DH)


def _head_mask():
    r = jax.lax.broadcasted_iota(jnp.int32, (E, E), 0) // DH
    c = jax.lax.broadcasted_iota(jnp.int32, (E, E), 1) // DH
    return (r == c).astype(jnp.float32)


def _encoder_kernel(xc_ref, xct_ref, z_ref, on_ref, lat_ref, wq_ref, wk_ref,
                    wv_ref, wo_ref, out_ref, qm_ref, eon_ref, von_ref,
                    num_ref, den_ref):
    u = pl.program_id(1)
    mhead = _head_mask()

    @pl.when(u == 0)
    def _init():
        qm = jnp.dot(lat_ref[...], wq_ref[...],
                     preferred_element_type=jnp.float32)
        qm_ref[...] = qm
        on = on_ref[0]
        kon = jnp.dot(on, wk_ref[...], preferred_element_type=jnp.float32)
        von_ref[...] = jnp.dot(on, wv_ref[...],
                               preferred_element_type=jnp.float32)
        son = jnp.dot(qm * kon, mhead,
                      preferred_element_type=jnp.float32) * INV_SQRT_DH
        eon_ref[...] = jnp.exp(son)
        num_ref[...] = jnp.zeros_like(num_ref)
        den_ref[...] = jnp.zeros_like(den_ref)

    xc = xc_ref[0]
    z = z_ref[0]
    gi = jnp.clip(jnp.floor(xc[:, 0:1] * (GH - 1) + 0.5), 0, GH - 1)
    gj = jnp.clip(jnp.floor(xc[:, 1:2] * (GW - 1) + 0.5), 0, GW - 1)
    idx = (gi * GW + gj).astype(jnp.int32)
    onehot = (idx == jax.lax.broadcasted_iota(jnp.int32, (BU, S), 1)
              ).astype(jnp.float32)

    xv = xct_ref[0]
    gi_r = jnp.clip(jnp.floor(xv[0:1, :] * (GH - 1) + 0.5), 0, GH - 1)
    gj_r = jnp.clip(jnp.floor(xv[1:2, :] * (GW - 1) + 0.5), 0, GW - 1)
    idx_r = (gi_r * GW + gj_r).astype(jnp.int32)
    onehot_t = (idx_r == jax.lax.broadcasted_iota(jnp.int32, (S, BU), 0)
                ).astype(jnp.float32)

    k = jnp.dot(z, wk_ref[...], preferred_element_type=jnp.float32)
    v = jnp.dot(z, wv_ref[...], preferred_element_type=jnp.float32)
    qg = jnp.dot(onehot, qm_ref[...], preferred_element_type=jnp.float32)
    scores = jnp.dot(qg * k, mhead,
                     preferred_element_type=jnp.float32) * INV_SQRT_DH
    w = jnp.exp(scores)

    num_ref[...] += jnp.dot(onehot_t, v * w, preferred_element_type=jnp.float32)
    den_ref[...] += jnp.dot(onehot_t, w, preferred_element_type=jnp.float32)

    @pl.when(u == NU - 1)
    def _finalize():
        eon = eon_ref[...]
        outm = (num_ref[...] + eon * von_ref[...]) / (den_ref[...] + eon)
        out_ref[0] = jnp.dot(outm, wo_ref[...],
                             preferred_element_type=jnp.float32)


def kernel(xc_off_grid, xc_on_grid, zc_off_grid, zc_on_grid, ignore_on_grid,
           latents, fake_embedding, Wq, Wk, Wv, Wo):
    Bv = xc_on_grid.shape[0]
    grid_shape = xc_on_grid.shape[1:-1]
    zc_on = zc_on_grid.reshape(Bv, S, E)
    on_tok = jnp.where(jnp.asarray(ignore_on_grid),
                       jnp.broadcast_to(fake_embedding, (Bv, S, E)), zc_on)

    out = pl.pallas_call(
        _encoder_kernel,
        grid=(Bv, NU),
        in_specs=[
            pl.BlockSpec((1, BU, DX), lambda b, u: (b, u, 0)),
            pl.BlockSpec((1, DX, BU), lambda b, u: (b, 0, u)),
            pl.BlockSpec((1, BU, E), lambda b, u: (b, u, 0)),
            pl.BlockSpec((1, S, E), lambda b, u: (b, 0, 0)),
            pl.BlockSpec((S, E), lambda b, u: (0, 0)),
            pl.BlockSpec((E, E), lambda b, u: (0, 0)),
            pl.BlockSpec((E, E), lambda b, u: (0, 0)),
            pl.BlockSpec((E, E), lambda b, u: (0, 0)),
            pl.BlockSpec((E, E), lambda b, u: (0, 0)),
        ],
        out_specs=pl.BlockSpec((1, S, E), lambda b, u: (b, 0, 0)),
        out_shape=jax.ShapeDtypeStruct((Bv, S, E), jnp.float32),
        scratch_shapes=[
            pltpu.VMEM((S, E), jnp.float32),
            pltpu.VMEM((S, E), jnp.float32),
            pltpu.VMEM((S, E), jnp.float32),
            pltpu.VMEM((S, E), jnp.float32),
            pltpu.VMEM((S, E), jnp.float32),
        ],
        compiler_params=pltpu.CompilerParams(
            dimension_semantics=("parallel", "arbitrary")),
    )(xc_off_grid, xc_off_grid.transpose(0, 2, 1), zc_off_grid, on_tok,
      latents, Wq, Wk, Wv, Wo)

    return out.reshape((Bv,) + tuple(grid_shape) + (E,))


# speedup vs baseline: 1.2595x; 1.0903x over previous
import jax
import jax.numpy as jnp
import numpy as np
from jax.experimental import pallas as pl
from jax.experimental.pallas import tpu as pltpu

B, U, GH, GW, E, DX, H = 4, 8192, 32, 32, 128, 2, 8
S = GH * GW
DH = E // H
BU = 2048
NU = U // BU
INV_SQRT_DH = 1.0 / np.sqrt(DH)


def _head_mask():
    r = jax.lax.broadcasted_iota(jnp.int32, (E, E), 0) // DH
    c = jax.lax.broadcasted_iota(jnp.int32, (E, E), 1) // DH
    return (r == c).astype(jnp.float32)


def _encoder_kernel(xc_ref, xct_ref, z_ref, on_ref, lat_ref, wq_ref, wk_ref,
                    wv_ref, wo_ref, out_ref, qm_ref, eon_ref, von_ref,
                    acc_ref, wkv_ref):
    u = pl.program_id(1)
    mhead = _head_mask()

    @pl.when(u == 0)
    def _init():
        wkv_ref[...] = jnp.concatenate([wk_ref[...], wv_ref[...]], axis=1)
        qm = jnp.dot(lat_ref[...], wq_ref[...],
                     preferred_element_type=jnp.float32)
        qm_ref[...] = qm
        on = on_ref[0]
        konvon = jnp.dot(on, wkv_ref[...], preferred_element_type=jnp.float32)
        von_ref[...] = konvon[:, E:]
        son = jnp.dot(qm * konvon[:, :E], mhead,
                      preferred_element_type=jnp.float32) * INV_SQRT_DH
        eon_ref[...] = jnp.exp(son)
        acc_ref[...] = jnp.zeros_like(acc_ref)

    xc = xc_ref[0]
    z = z_ref[0]
    gi = jnp.clip(jnp.floor(xc[:, 0:1] * (GH - 1) + 0.5), 0, GH - 1)
    gj = jnp.clip(jnp.floor(xc[:, 1:2] * (GW - 1) + 0.5), 0, GW - 1)
    idx = (gi * GW + gj).astype(jnp.int32)
    onehot = (idx == jax.lax.broadcasted_iota(jnp.int32, (BU, S), 1)
              ).astype(jnp.float32)

    xv = xct_ref[0]
    gi_r = jnp.clip(jnp.floor(xv[0:1, :] * (GH - 1) + 0.5), 0, GH - 1)
    gj_r = jnp.clip(jnp.floor(xv[1:2, :] * (GW - 1) + 0.5), 0, GW - 1)
    idx_r = (gi_r * GW + gj_r).astype(jnp.int32)
    onehot_t = (idx_r == jax.lax.broadcasted_iota(jnp.int32, (S, BU), 0)
                ).astype(jnp.float32)

    kv = jnp.dot(z, wkv_ref[...], preferred_element_type=jnp.float32)
    k, v = kv[:, :E], kv[:, E:]
    qg = jnp.dot(onehot, qm_ref[...], preferred_element_type=jnp.float32)
    scores = jnp.dot(qg * k, mhead,
                     preferred_element_type=jnp.float32) * INV_SQRT_DH
    w = jnp.exp(scores)

    payload = jnp.concatenate([v * w, w], axis=1)
    acc_ref[...] += jnp.dot(onehot_t, payload,
                            preferred_element_type=jnp.float32)

    @pl.when(u == NU - 1)
    def _finalize():
        eon = eon_ref[...]
        num, den = acc_ref[:, :E], acc_ref[:, E:]
        outm = (num + eon * von_ref[...]) / (den + eon)
        out_ref[0] = jnp.dot(outm, wo_ref[...],
                             preferred_element_type=jnp.float32)


def kernel(xc_off_grid, xc_on_grid, zc_off_grid, zc_on_grid, ignore_on_grid,
           latents, fake_embedding, Wq, Wk, Wv, Wo):
    Bv = xc_on_grid.shape[0]
    grid_shape = xc_on_grid.shape[1:-1]
    zc_on = zc_on_grid.reshape(Bv, S, E)
    on_tok = jnp.where(jnp.asarray(ignore_on_grid),
                       jnp.broadcast_to(fake_embedding, (Bv, S, E)), zc_on)

    out = pl.pallas_call(
        _encoder_kernel,
        grid=(Bv, NU),
        in_specs=[
            pl.BlockSpec((1, BU, DX), lambda b, u: (b, u, 0)),
            pl.BlockSpec((1, DX, BU), lambda b, u: (b, 0, u)),
            pl.BlockSpec((1, BU, E), lambda b, u: (b, u, 0)),
            pl.BlockSpec((1, S, E), lambda b, u: (b, 0, 0)),
            pl.BlockSpec((S, E), lambda b, u: (0, 0)),
            pl.BlockSpec((E, E), lambda b, u: (0, 0)),
            pl.BlockSpec((E, E), lambda b, u: (0, 0)),
            pl.BlockSpec((E, E), lambda b, u: (0, 0)),
            pl.BlockSpec((E, E), lambda b, u: (0, 0)),
        ],
        out_specs=pl.BlockSpec((1, S, E), lambda b, u: (b, 0, 0)),
        out_shape=jax.ShapeDtypeStruct((Bv, S, E), jnp.float32),
        scratch_shapes=[
            pltpu.VMEM((S, E), jnp.float32),
            pltpu.VMEM((S, E), jnp.float32),
            pltpu.VMEM((S, E), jnp.float32),
            pltpu.VMEM((S, 2 * E), jnp.float32),
            pltpu.VMEM((E, 2 * E), jnp.float32),
        ],
        compiler_params=pltpu.CompilerParams(
            dimension_semantics=("parallel", "arbitrary")),
    )(xc_off_grid, xc_off_grid.transpose(0, 2, 1), zc_off_grid, on_tok,
      latents, Wq, Wk, Wv, Wo)

    return out.reshape((Bv,) + tuple(grid_shape) + (E,))
